# Initial kernel scaffold; baseline (speedup 1.0000x reference)
import jax
import jax.numpy as jnp
from jax.experimental import pallas as pl


def kernel(x, q, W_bb, b_bb, W1, b1, W2, b2, W3, b3):
    raise NotImplementedError("write your pallas kernel here")


# speedup vs baseline: 11.2389x; 11.2389x over previous
import numpy as np
import jax
import jax.numpy as jnp
from jax.experimental import pallas as pl
from jax.experimental.pallas import tpu as pltpu

_QL = (0.025, 0.1, 0.25, 0.5, 0.75, 0.9, 0.975)
_B, _T, _D = 8192, 512, 512
_H1, _H2, _HOR = 256, 128, 96
_QF, _QT = 7, 32
_BB = 64


def _bitonic_sort_lanes(v, n):
    ii = jax.lax.broadcasted_iota(jnp.int32, v.shape, v.ndim - 1)
    k = 2
    while k <= n:
        d = k // 2
        while d >= 1:
            lower = (ii & d) == 0
            partner = jnp.where(lower,
                                jnp.roll(v, -d, axis=-1),
                                jnp.roll(v, d, axis=-1))
            up = (ii & k) == 0
            take_min = up == lower
            v = jnp.where(take_min,
                          jnp.minimum(v, partner),
                          jnp.maximum(v, partner))
            d //= 2
        k *= 2
    return v


def _sort7(vals):
    vals = list(vals)
    for r in range(7):
        for i in range(r % 2, 6, 2):
            a, b = vals[i], vals[i + 1]
            vals[i] = jnp.minimum(a, b)
            vals[i + 1] = jnp.maximum(a, b)
    return vals


def _fused_kernel(x_ref, q_ref, wbb_ref, bbb_ref, w1_ref, b1_ref,
                  w2_ref, b2_ref, w3_ref, b3_ref, out_ref):
    feats = jnp.dot(x_ref[...], wbb_ref[...],
                    preferred_element_type=jnp.float32) + bbb_ref[...]

    heads = []
    for j in range(_QF):
        h = jnp.dot(feats, w1_ref[j], preferred_element_type=jnp.float32)
        h = jnp.maximum(h + b1_ref[j:j + 1, :], 0.0)
        h = jnp.dot(h, w2_ref[j], preferred_element_type=jnp.float32)
        h = jnp.maximum(h + b2_ref[j:j + 1, :], 0.0)
        h = jnp.dot(h, w3_ref[j], preferred_element_type=jnp.float32)
        heads.append(h + b3_ref[j:j + 1, :])
    s = _sort7(heads)

    qs = _bitonic_sort_lanes(q_ref[...], _QT)

    cnt = jnp.zeros_like(qs)
    for lev in _QL:
        cnt = cnt + jnp.where(qs > lev, 1.0, 0.0)
    idx_high = jnp.clip(cnt, 1.0, float(_QF - 1))

    q_low = jnp.full_like(qs, _QL[0])
    q_high = jnp.full_like(qs, _QL[1])
    for j in range(2, _QF):
        sel = idx_high == float(j)
        q_low = jnp.where(sel, _QL[j - 1], q_low)
        q_high = jnp.where(sel, _QL[j], q_high)

    w = (qs - q_low) / (q_high - q_low + 1e-8)
    w = jnp.where(qs <= _QL[0], 0.0, jnp.where(qs >= _QL[-1], 1.0, w))

    acc = jnp.zeros((x_ref.shape[0], _HOR, _QT), dtype=jnp.float32)
    one_m_w = 1.0 - w
    for j in range(_QF):
        c = jnp.where(idx_high == float(j + 1), one_m_w, 0.0)
        if j > 0:
            c = c + jnp.where(idx_high == float(j), w, 0.0)
        acc = acc + s[j][:, :, None] * c[:, None, :]
    out_ref[...] = acc


def kernel(x, q, W_bb, b_bb, W1, b1, W2, b2, W3, b3):
    grid = (_B // _BB,)
    out = pl.pallas_call(
        _fused_kernel,
        out_shape=jax.ShapeDtypeStruct((_B, _HOR, _QT), jnp.float32),
        grid=grid,
        in_specs=[
            pl.BlockSpec((_BB, _T), lambda i: (i, 0)),
            pl.BlockSpec((_BB, _QT), lambda i: (i, 0)),
            pl.BlockSpec((_T, _D), lambda i: (0, 0)),
            pl.BlockSpec((1, _D), lambda i: (0, 0)),
            pl.BlockSpec((_QF, _D, _H1), lambda i: (0, 0, 0)),
            pl.BlockSpec((_QF, _H1), lambda i: (0, 0)),
            pl.BlockSpec((_QF, _H1, _H2), lambda i: (0, 0, 0)),
            pl.BlockSpec((_QF, _H2), lambda i: (0, 0)),
            pl.BlockSpec((_QF, _H2, _HOR), lambda i: (0, 0, 0)),
            pl.BlockSpec((_QF, _HOR), lambda i: (0, 0)),
        ],
        out_specs=pl.BlockSpec((_BB, _HOR, _QT), lambda i: (i, 0, 0)),
        compiler_params=pltpu.CompilerParams(
            dimension_semantics=("parallel",),
            vmem_limit_bytes=56 * 1024 * 1024,
        ),
        name="mhq_nbeats_fused",
    )(x, q, W_bb, b_bb.reshape(1, _D), W1, b1, W2, b2, W3, b3)
    return out


# speedup vs baseline: 17.8915x; 1.5919x over previous
import numpy as np
import jax
import jax.numpy as jnp
from jax.experimental import pallas as pl
from jax.experimental.pallas import tpu as pltpu

_QL = (0.025, 0.1, 0.25, 0.5, 0.75, 0.9, 0.975)
_B, _T, _D = 8192, 512, 512
_H1, _H2, _HOR = 256, 128, 96
_QF, _QT = 7, 32
_BB = 64


def _bitonic_sort_lanes(v, n):
    ii = jax.lax.broadcasted_iota(jnp.int32, v.shape, v.ndim - 1)
    k = 2
    while k <= n:
        d = k // 2
        while d >= 1:
            lower = (ii & d) == 0
            partner = jnp.where(lower,
                                jnp.roll(v, -d, axis=-1),
                                jnp.roll(v, d, axis=-1))
            up = (ii & k) == 0
            take_min = up == lower
            v = jnp.where(take_min,
                          jnp.minimum(v, partner),
                          jnp.maximum(v, partner))
            d //= 2
        k *= 2
    return v


def _sort7(vals):
    vals = list(vals)
    for r in range(7):
        for i in range(r % 2, 6, 2):
            a, b = vals[i], vals[i + 1]
            vals[i] = jnp.minimum(a, b)
            vals[i + 1] = jnp.maximum(a, b)
    return vals


def _fused_kernel(x_ref, q_ref, wbb_ref, bbb_ref, w1_ref, b1_ref,
                  w2_ref, b2_ref, w3_ref, b3_ref, out_ref):
    feats = jnp.dot(x_ref[...], wbb_ref[...],
                    preferred_element_type=jnp.float32) + bbb_ref[...]

    heads = []
    for j in range(_QF):
        h = jnp.dot(feats, w1_ref[j], preferred_element_type=jnp.float32)
        h = jnp.maximum(h + b1_ref[j:j + 1, :], 0.0)
        h = jnp.dot(h, w2_ref[j], preferred_element_type=jnp.float32)
        h = jnp.maximum(h + b2_ref[j:j + 1, :], 0.0)
        h = jnp.dot(h, w3_ref[j], preferred_element_type=jnp.float32)
        heads.append(h + b3_ref[j:j + 1, :])
    s = _sort7(heads)

    qs = _bitonic_sort_lanes(q_ref[...], _QT)

    cnt = jnp.zeros_like(qs)
    for lev in _QL:
        cnt = cnt + jnp.where(qs > lev, 1.0, 0.0)
    idx_high = jnp.clip(cnt, 1.0, float(_QF - 1))

    q_low = jnp.full_like(qs, _QL[0])
    q_high = jnp.full_like(qs, _QL[1])
    for j in range(2, _QF):
        sel = idx_high == float(j)
        q_low = jnp.where(sel, _QL[j - 1], q_low)
        q_high = jnp.where(sel, _QL[j], q_high)

    w = (qs - q_low) / (q_high - q_low + 1e-8)
    w = jnp.where(qs <= _QL[0], 0.0, jnp.where(qs >= _QL[-1], 1.0, w))

    acc = jnp.zeros((x_ref.shape[0], _QT, _HOR), dtype=jnp.float32)
    one_m_w = 1.0 - w
    for j in range(_QF):
        c = jnp.where(idx_high == float(j + 1), one_m_w, 0.0)
        if j > 0:
            c = c + jnp.where(idx_high == float(j), w, 0.0)
        acc = acc + c[:, :, None] * s[j][:, None, :]
    out_ref[...] = jnp.swapaxes(acc, 1, 2)


def kernel(x, q, W_bb, b_bb, W1, b1, W2, b2, W3, b3):
    grid = (_B // _BB,)
    out = pl.pallas_call(
        _fused_kernel,
        out_shape=jax.ShapeDtypeStruct((_B, _HOR, _QT), jnp.float32),
        grid=grid,
        in_specs=[
            pl.BlockSpec((_BB, _T), lambda i: (i, 0)),
            pl.BlockSpec((_BB, _QT), lambda i: (i, 0)),
            pl.BlockSpec((_T, _D), lambda i: (0, 0)),
            pl.BlockSpec((1, _D), lambda i: (0, 0)),
            pl.BlockSpec((_QF, _D, _H1), lambda i: (0, 0, 0)),
            pl.BlockSpec((_QF, _H1), lambda i: (0, 0)),
            pl.BlockSpec((_QF, _H1, _H2), lambda i: (0, 0, 0)),
            pl.BlockSpec((_QF, _H2), lambda i: (0, 0)),
            pl.BlockSpec((_QF, _H2, _HOR), lambda i: (0, 0, 0)),
            pl.BlockSpec((_QF, _HOR), lambda i: (0, 0)),
        ],
        out_specs=pl.BlockSpec((_BB, _HOR, _QT), lambda i: (i, 0, 0)),
        compiler_params=pltpu.CompilerParams(
            dimension_semantics=("parallel",),
            vmem_limit_bytes=56 * 1024 * 1024,
        ),
        name="mhq_nbeats_fused",
    )(x, q, W_bb, b_bb.reshape(1, _D), W1, b1, W2, b2, W3, b3)
    return out


# speedup vs baseline: 17.9065x; 1.0008x over previous
import numpy as np
import jax
import jax.numpy as jnp
from jax.experimental import pallas as pl
from jax.experimental.pallas import tpu as pltpu

_QL = (0.025, 0.1, 0.25, 0.5, 0.75, 0.9, 0.975)
_B, _T, _D = 8192, 512, 512
_H1, _H2, _HOR = 256, 128, 96
_QF, _QT = 7, 32
_BB = 64


def _bitonic_sort_lanes(v, n):
    ii = jax.lax.broadcasted_iota(jnp.int32, v.shape, v.ndim - 1)
    k = 2
    while k <= n:
        d = k // 2
        while d >= 1:
            lower = (ii & d) == 0
            partner = jnp.where(lower,
                                jnp.roll(v, -d, axis=-1),
                                jnp.roll(v, d, axis=-1))
            up = (ii & k) == 0
            take_min = up == lower
            v = jnp.where(take_min,
                          jnp.minimum(v, partner),
                          jnp.maximum(v, partner))
            d //= 2
        k *= 2
    return v


def _sort7(vals):
    vals = list(vals)
    for r in range(7):
        for i in range(r % 2, 6, 2):
            a, b = vals[i], vals[i + 1]
            vals[i] = jnp.minimum(a, b)
            vals[i + 1] = jnp.maximum(a, b)
    return vals


def _fused_kernel(x_ref, q_ref, wbb_ref, bbb_ref, w1_ref, b1_ref,
                  w2_ref, b2_ref, w3_ref, b3_ref, out_ref):
    feats = jnp.dot(x_ref[...], wbb_ref[...],
                    preferred_element_type=jnp.float32) + bbb_ref[...]

    heads = []
    for j in range(_QF):
        h = jnp.dot(feats, w1_ref[j], preferred_element_type=jnp.float32)
        h = jnp.maximum(h + b1_ref[j:j + 1, :], 0.0)
        h = jnp.dot(h, w2_ref[j], preferred_element_type=jnp.float32)
        h = jnp.maximum(h + b2_ref[j:j + 1, :], 0.0)
        h = jnp.dot(h, w3_ref[j], preferred_element_type=jnp.float32)
        heads.append(h + b3_ref[j:j + 1, :])
    s = _sort7(heads)

    qs = _bitonic_sort_lanes(q_ref[...], _QT)

    cnt = jnp.zeros_like(qs)
    for lev in _QL:
        cnt = cnt + jnp.where(qs > lev, 1.0, 0.0)
    idx_high = jnp.clip(cnt, 1.0, float(_QF - 1))

    q_low = jnp.full_like(qs, _QL[0])
    q_high = jnp.full_like(qs, _QL[1])
    for j in range(2, _QF):
        sel = idx_high == float(j)
        q_low = jnp.where(sel, _QL[j - 1], q_low)
        q_high = jnp.where(sel, _QL[j], q_high)

    w = (qs - q_low) / (q_high - q_low + 1e-8)
    w = jnp.where(qs <= _QL[0], 0.0, jnp.where(qs >= _QL[-1], 1.0, w))

    one_m_w = 1.0 - w
    cs = []
    for j in range(_QF):
        c = jnp.where(idx_high == float(j + 1), one_m_w, 0.0)
        if j > 0:
            c = c + jnp.where(idx_high == float(j), w, 0.0)
        cs.append(c)

    rc = 16
    for b0 in range(0, _BB, rc):
        acc = jnp.zeros((rc, _QT, _HOR), dtype=jnp.float32)
        for j in range(_QF):
            acc = acc + (cs[j][b0:b0 + rc, :, None]
                         * s[j][b0:b0 + rc, None, :])
        out_ref[b0:b0 + rc] = jnp.swapaxes(acc, 1, 2)


def kernel(x, q, W_bb, b_bb, W1, b1, W2, b2, W3, b3):
    grid = (_B // _BB,)
    out = pl.pallas_call(
        _fused_kernel,
        out_shape=jax.ShapeDtypeStruct((_B, _HOR, _QT), jnp.float32),
        grid=grid,
        in_specs=[
            pl.BlockSpec((_BB, _T), lambda i: (i, 0)),
            pl.BlockSpec((_BB, _QT), lambda i: (i, 0)),
            pl.BlockSpec((_T, _D), lambda i: (0, 0)),
            pl.BlockSpec((1, _D), lambda i: (0, 0)),
            pl.BlockSpec((_QF, _D, _H1), lambda i: (0, 0, 0)),
            pl.BlockSpec((_QF, _H1), lambda i: (0, 0)),
            pl.BlockSpec((_QF, _H1, _H2), lambda i: (0, 0, 0)),
            pl.BlockSpec((_QF, _H2), lambda i: (0, 0)),
            pl.BlockSpec((_QF, _H2, _HOR), lambda i: (0, 0, 0)),
            pl.BlockSpec((_QF, _HOR), lambda i: (0, 0)),
        ],
        out_specs=pl.BlockSpec((_BB, _HOR, _QT), lambda i: (i, 0, 0)),
        compiler_params=pltpu.CompilerParams(
            dimension_semantics=("parallel",),
            vmem_limit_bytes=56 * 1024 * 1024,
        ),
        name="mhq_nbeats_fused",
    )(x, q, W_bb, b_bb.reshape(1, _D), W1, b1, W2, b2, W3, b3)
    return out
